# Initial kernel scaffold; baseline (speedup 1.0000x reference)
import jax
import jax.numpy as jnp
from jax.experimental import pallas as pl


def kernel(node_features, frac_coords, lattices, edge_index, edge2graph, frac_diff, We1, be1, We2, be2, Wn1, bn1, Wn2, bn2):
    raise NotImplementedError("write your pallas kernel here")


# speedup vs baseline: 2.9576x; 2.9576x over previous
import functools

import jax
import jax.numpy as jnp
from jax import lax
from jax.experimental import pallas as pl
from jax.experimental.pallas import tpu as pltpu
from jax.experimental.pallas import tpu_sc as plsc

N = 10000
E = 320000
G = 256
H = 128

NW = 32
ET = E // NW
CH = 80
NCH = ET // CH
NP = 10240
NT = NP // 16
WCH = 128
NWCH = NT // WCH

def _mesh():
    return plsc.VectorSubcoreMesh(core_axis_name="c", subcore_axis_name="s")


def _silu(x):
    return x * jax.nn.sigmoid(x)


def _p1_body(nf, latp, whi_t, whj_t, wlat_t, be1, a_out, b_out, c_out):
    x = nf[...]
    a_out[...] = jnp.dot(x, whi_t[...], preferred_element_type=jnp.float32)
    b_out[...] = jnp.dot(x, whj_t[...], preferred_element_type=jnp.float32)
    c_out[...] = (
        jnp.dot(latp[...], wlat_t[...], preferred_element_type=jnp.float32)
        + be1[...]
    )


def _p1(nf, latp, whi_t, whj_t, wlat_t, be1):
    return pl.pallas_call(
        _p1_body,
        out_shape=(
            jax.ShapeDtypeStruct((N, H), jnp.float32),
            jax.ShapeDtypeStruct((N, H), jnp.float32),
            jax.ShapeDtypeStruct((G, H), jnp.float32),
        ),
    )(nf, latp, whi_t, whj_t, wlat_t, be1)


def _p2_body(a_hbm, b_hbm, c_hbm, src_hbm, dst_hbm, g_hbm, out_hbm,
             isrc, idst, ig, buf_a, buf_b, buf_c, s1, s2, s3):
    c = lax.axis_index("c")
    s = lax.axis_index("s")
    wid = s * 2 + c
    base = wid * ET

    def chunk(ci, carry):
        off = pl.multiple_of(base + ci * CH, 8)
        pltpu.sync_copy(src_hbm.at[pl.ds(off, CH)], isrc)
        pltpu.sync_copy(dst_hbm.at[pl.ds(off, CH)], idst)
        pltpu.sync_copy(g_hbm.at[pl.ds(off, CH)], ig)
        ca = pltpu.async_copy(a_hbm.at[isrc], buf_a, s1)
        cb = pltpu.async_copy(b_hbm.at[idst], buf_b, s2)
        cc = pltpu.async_copy(c_hbm.at[ig], buf_c, s3)
        ca.wait()
        cb.wait()
        cc.wait()

        def add_row(r, c2):
            for j in range(H // 16):
                sl = pl.ds(j * 16, 16)
                buf_a[r, sl] = buf_a[r, sl] + buf_b[r, sl] + buf_c[r, sl]
            return c2

        lax.fori_loop(0, CH, add_row, 0)
        pltpu.sync_copy(buf_a, out_hbm.at[pl.ds(off, CH)])
        return carry

    lax.fori_loop(0, NCH, chunk, 0)


def _p2(a, b, c, src, dst, g):
    f = functools.partial(
        pl.kernel,
        out_type=jax.ShapeDtypeStruct((E, H), jnp.float32),
        mesh=_mesh(),
        scratch_types=[
            pltpu.VMEM((CH,), jnp.int32),
            pltpu.VMEM((CH,), jnp.int32),
            pltpu.VMEM((CH,), jnp.int32),
            pltpu.VMEM((CH, H), jnp.float32),
            pltpu.VMEM((CH, H), jnp.float32),
            pltpu.VMEM((CH, H), jnp.float32),
            pltpu.SemaphoreType.DMA,
            pltpu.SemaphoreType.DMA,
            pltpu.SemaphoreType.DMA,
        ],
    )(_p2_body)
    return f(a, b, c, src, dst, g)


def _p2c_body(src_hbm, cnt_hbm, isrc, ones_b, z_b, cacc):
    c = lax.axis_index("c")
    s = lax.axis_index("s")
    base = c * (E // 2) + s * ET

    def orow(r, c2):
        for j in range(H // 16):
            ones_b[r, pl.ds(j * 16, 16)] = jnp.ones((16,), jnp.float32)
        return c2

    lax.fori_loop(0, CH, orow, 0)

    def zrow(r, c2):
        for j in range(H // 16):
            z_b[r, pl.ds(j * 16, 16)] = jnp.zeros((16,), jnp.float32)
        return c2

    lax.fori_loop(0, WCH, zrow, 0)

    for j in range(NWCH):
        r0 = pl.multiple_of(s * NT + j * WCH, 8)
        pltpu.sync_copy(z_b, cacc.at[pl.ds(r0, WCH)])

    plsc.subcore_barrier()

    def chunk(ci, carry):
        off = pl.multiple_of(base + ci * CH, 8)
        pltpu.sync_copy(src_hbm.at[pl.ds(off, CH)], isrc)
        pltpu.sync_copy(ones_b, cacc.at[isrc], add=True)
        return carry

    lax.fori_loop(0, NCH, chunk, 0)

    plsc.subcore_barrier()

    for j in range(NWCH):
        r0 = pl.multiple_of(s * NT + j * WCH, 8)
        o0 = pl.multiple_of(c * NP + r0, 8)
        pltpu.sync_copy(cacc.at[pl.ds(r0, WCH)], z_b)
        pltpu.sync_copy(z_b, cnt_hbm.at[pl.ds(o0, WCH)])


def _p2c(src):
    f = functools.partial(
        pl.kernel,
        out_type=jax.ShapeDtypeStruct((2 * NP, H), jnp.float32),
        mesh=_mesh(),
        scratch_types=[
            pltpu.VMEM((CH,), jnp.int32),
            pltpu.VMEM((CH, H), jnp.float32),
            pltpu.VMEM((WCH, H), jnp.float32),
            pltpu.VMEM_SHARED((NP, H), jnp.float32),
        ],
    )(_p2c_body)
    return f(src)


BE = 2560


def _p3_body(pre, fdp, wfd_t, we2_t, be2, out):
    x = pre[...] + jnp.dot(fdp[...], wfd_t[...],
                           preferred_element_type=jnp.float32)
    e1 = _silu(x)
    y = jnp.dot(e1, we2_t[...], preferred_element_type=jnp.float32) + be2[...]
    out[...] = _silu(y)


def _p3(pre, fdp, wfd_t, we2_t, be2):
    return pl.pallas_call(
        _p3_body,
        grid=(E // BE,),
        in_specs=[
            pl.BlockSpec((BE, H), lambda i: (i, 0)),
            pl.BlockSpec((BE, 8), lambda i: (i, 0)),
            pl.BlockSpec((8, H), lambda i: (0, 0)),
            pl.BlockSpec((H, H), lambda i: (0, 0)),
            pl.BlockSpec((1, H), lambda i: (0, 0)),
        ],
        out_specs=pl.BlockSpec((BE, H), lambda i: (i, 0)),
        out_shape=jax.ShapeDtypeStruct((E, H), jnp.float32),
    )(pre, fdp, wfd_t, we2_t, be2)


def _p4_body(ef_hbm, src_hbm, sum_hbm, idx, buf_e, wout, acc):
    c = lax.axis_index("c")
    s = lax.axis_index("s")
    base = c * (E // 2) + s * ET

    def zrow(r, c2):
        for j in range(H // 16):
            wout[r, pl.ds(j * 16, 16)] = jnp.zeros((16,), jnp.float32)
        return c2

    lax.fori_loop(0, WCH, zrow, 0)

    for j in range(NWCH):
        r0 = pl.multiple_of(s * NT + j * WCH, 8)
        pltpu.sync_copy(wout, acc.at[pl.ds(r0, WCH)])

    plsc.subcore_barrier()

    def chunk(ci, carry):
        off = pl.multiple_of(base + ci * CH, 8)
        pltpu.sync_copy(src_hbm.at[pl.ds(off, CH)], idx)
        pltpu.sync_copy(ef_hbm.at[pl.ds(off, CH)], buf_e)
        pltpu.sync_copy(buf_e, acc.at[idx], add=True)
        return carry

    lax.fori_loop(0, NCH, chunk, 0)

    plsc.subcore_barrier()

    for j in range(NWCH):
        r0 = pl.multiple_of(s * NT + j * WCH, 8)
        o0 = pl.multiple_of(c * NP + r0, 8)
        pltpu.sync_copy(acc.at[pl.ds(r0, WCH)], wout)
        pltpu.sync_copy(wout, sum_hbm.at[pl.ds(o0, WCH)])


def _p4(ef, src):
    f = functools.partial(
        pl.kernel,
        out_type=jax.ShapeDtypeStruct((2 * NP, H), jnp.float32),
        mesh=_mesh(),
        scratch_types=[
            pltpu.VMEM((CH,), jnp.int32),
            pltpu.VMEM((CH, H), jnp.float32),
            pltpu.VMEM((WCH, H), jnp.float32),
            pltpu.VMEM_SHARED((NP, H), jnp.float32),
        ],
    )(_p4_body)
    return f(ef, src)


BN = 2000


def _p5_body(nf, sum0, sum1, cnt0, cnt1, wn1a_t, wn1b_t, bn1, wn2_t, bn2,
             out):
    x = nf[...]
    ssum = sum0[0] + sum1[0]
    cnt = cnt0[0][:, 0:1] + cnt1[0][:, 0:1]
    agg = ssum / jnp.clip(cnt, 1.0, None)
    h = (
        jnp.dot(x, wn1a_t[...], preferred_element_type=jnp.float32)
        + jnp.dot(agg, wn1b_t[...], preferred_element_type=jnp.float32)
        + bn1[...]
    )
    h = _silu(h)
    y = jnp.dot(h, wn2_t[...], preferred_element_type=jnp.float32) + bn2[...]
    out[...] = x + _silu(y)


def _p5(nf, sums, cnts, wn1a_t, wn1b_t, bn1, wn2_t, bn2):
    nb = N // BN
    return pl.pallas_call(
        _p5_body,
        grid=(nb,),
        in_specs=[
            pl.BlockSpec((BN, H), lambda i: (i, 0)),
            pl.BlockSpec((1, BN, H), lambda i: (0, i, 0)),
            pl.BlockSpec((1, BN, H), lambda i: (1, i, 0)),
            pl.BlockSpec((1, BN, H), lambda i: (0, i, 0)),
            pl.BlockSpec((1, BN, H), lambda i: (1, i, 0)),
            pl.BlockSpec((H, H), lambda i: (0, 0)),
            pl.BlockSpec((H, H), lambda i: (0, 0)),
            pl.BlockSpec((1, H), lambda i: (0, 0)),
            pl.BlockSpec((H, H), lambda i: (0, 0)),
            pl.BlockSpec((1, H), lambda i: (0, 0)),
        ],
        out_specs=pl.BlockSpec((BN, H), lambda i: (i, 0)),
        out_shape=jax.ShapeDtypeStruct((N, H), jnp.float32),
    )(nf, sums, sums, cnts, cnts, wn1a_t, wn1b_t, bn1, wn2_t, bn2)


def kernel(node_features, frac_coords, lattices, edge_index, edge2graph,
           frac_diff, We1, be1, We2, be2, Wn1, bn1, Wn2, bn2):
    src = edge_index[0].astype(jnp.int32)
    dst = edge_index[1].astype(jnp.int32)
    g = edge2graph.astype(jnp.int32)

    whi_t = We1[:, :H].T
    whj_t = We1[:, H:2 * H].T
    wlat_t = jnp.pad(We1[:, 2 * H:2 * H + 6].T, ((0, 2), (0, 0)))
    wfd_t = jnp.pad(We1[:, 2 * H + 6:].T, ((0, 5), (0, 0)))
    latp = jnp.pad(lattices, ((0, 0), (0, 2)))
    fdp = jnp.pad(frac_diff, ((0, 0), (0, 5)))

    a, b, c = _p1(node_features, latp, whi_t, whj_t, wlat_t,
                  be1.reshape(1, H))
    pre = _p2(a, b, c, src, dst, g)
    cnts = _p2c(src)
    ef = _p3(pre, fdp, wfd_t, We2.T, be2.reshape(1, H))
    sums = _p4(ef, src)
    sums = sums.reshape(2, NP, H)
    cnts = cnts.reshape(2, NP, H)
    return _p5(node_features, sums, cnts, Wn1[:, :H].T, Wn1[:, H:].T,
               bn1.reshape(1, H), Wn2.T, bn2.reshape(1, H))


# speedup vs baseline: 3.5162x; 1.1889x over previous
import functools

import jax
import jax.numpy as jnp
from jax import lax
from jax.experimental import pallas as pl
from jax.experimental.pallas import tpu as pltpu
from jax.experimental.pallas import tpu_sc as plsc

N = 10000
E = 320000
G = 256
H = 128

NW = 32
ET = E // NW
CH = 80
NCH = ET // CH
NP = 10240
NT = NP // 16
WCH = 128
NWCH = NT // WCH

def _mesh():
    return plsc.VectorSubcoreMesh(core_axis_name="c", subcore_axis_name="s")


def _silu(x):
    return x * jax.nn.sigmoid(x)


def _p1_body(nf, latp, whi_t, whj_t, wlat_t, be1, a_out, b_out, c_out):
    x = nf[...]
    a_out[...] = jnp.dot(x, whi_t[...], preferred_element_type=jnp.float32)
    b_out[...] = jnp.dot(x, whj_t[...], preferred_element_type=jnp.float32)
    c_out[...] = (
        jnp.dot(latp[...], wlat_t[...], preferred_element_type=jnp.float32)
        + be1[...]
    )


def _p1(nf, latp, whi_t, whj_t, wlat_t, be1):
    return pl.pallas_call(
        _p1_body,
        out_shape=(
            jax.ShapeDtypeStruct((N, H), jnp.float32),
            jax.ShapeDtypeStruct((N, H), jnp.float32),
            jax.ShapeDtypeStruct((G, H), jnp.float32),
        ),
    )(nf, latp, whi_t, whj_t, wlat_t, be1)


def _p2_body(a_hbm, b_hbm, c_hbm, src_hbm, dst_hbm, g_hbm, out_hbm,
             isrc, idst, ig, bufs, sems):
    c = lax.axis_index("c")
    s = lax.axis_index("s")
    wid = s * 2 + c
    base = wid * ET

    pltpu.sync_copy(src_hbm.at[pl.ds(pl.multiple_of(base, 8), ET)], isrc)
    pltpu.sync_copy(dst_hbm.at[pl.ds(pl.multiple_of(base, 8), ET)], idst)
    pltpu.sync_copy(g_hbm.at[pl.ds(pl.multiple_of(base, 8), ET)], ig)

    def issue(ci, k):
        lo = pl.multiple_of(ci * CH, 8)
        pltpu.async_copy(a_hbm.at[isrc.at[pl.ds(lo, CH)]], bufs[k][0],
                         sems[k][0])
        pltpu.async_copy(b_hbm.at[idst.at[pl.ds(lo, CH)]], bufs[k][1],
                         sems[k][1])
        pltpu.async_copy(c_hbm.at[ig.at[pl.ds(lo, CH)]], bufs[k][2],
                         sems[k][2])

    def wait(k):
        for j in range(3):
            pltpu.make_async_copy(a_hbm.at[isrc.at[pl.ds(0, CH)]],
                                  bufs[k][j], sems[k][j]).wait()

    def process(ci, k):
        ba, bb, bc = bufs[k]

        def add_row(r, c2):
            for j in range(H // 16):
                sl = pl.ds(j * 16, 16)
                ba[r, sl] = ba[r, sl] + bb[r, sl] + bc[r, sl]
            return c2

        lax.fori_loop(0, CH, add_row, 0)
        off = pl.multiple_of(base + ci * CH, 8)
        pltpu.sync_copy(ba, out_hbm.at[pl.ds(off, CH)])

    issue(0, 0)

    def body2(kk, carry):
        ci = kk * 2
        issue(ci + 1, 1)
        wait(0)
        process(ci, 0)
        issue(ci + 2, 0)
        wait(1)
        process(ci + 1, 1)
        return carry

    lax.fori_loop(0, (NCH - 1) // 2, body2, 0)
    wait(0)
    process(NCH - 1, 0)


def _p2(a, b, c, src, dst, g):
    f = functools.partial(
        pl.kernel,
        out_type=jax.ShapeDtypeStruct((E, H), jnp.float32),
        mesh=_mesh(),
        scratch_types=[
            pltpu.VMEM((ET,), jnp.int32),
            pltpu.VMEM((ET,), jnp.int32),
            pltpu.VMEM((ET,), jnp.int32),
            [[pltpu.VMEM((CH, H), jnp.float32) for _ in range(3)]
             for _ in range(2)],
            [[pltpu.SemaphoreType.DMA for _ in range(3)] for _ in range(2)],
        ],
    )(_p2_body)
    return f(a, b, c, src, dst, g)


def _p2c_body(src_hbm, cnt_hbm, isrc, ones_b, z_b, cacc):
    c = lax.axis_index("c")
    s = lax.axis_index("s")
    base = c * (E // 2) + s * ET

    def orow(r, c2):
        for j in range(H // 16):
            ones_b[r, pl.ds(j * 16, 16)] = jnp.ones((16,), jnp.float32)
        return c2

    lax.fori_loop(0, CH, orow, 0)

    def zrow(r, c2):
        for j in range(H // 16):
            z_b[r, pl.ds(j * 16, 16)] = jnp.zeros((16,), jnp.float32)
        return c2

    lax.fori_loop(0, WCH, zrow, 0)

    for j in range(NWCH):
        r0 = pl.multiple_of(s * NT + j * WCH, 8)
        pltpu.sync_copy(z_b, cacc.at[pl.ds(r0, WCH)])

    plsc.subcore_barrier()

    def chunk(ci, carry):
        off = pl.multiple_of(base + ci * CH, 8)
        pltpu.sync_copy(src_hbm.at[pl.ds(off, CH)], isrc)
        pltpu.sync_copy(ones_b, cacc.at[isrc], add=True)
        return carry

    lax.fori_loop(0, NCH, chunk, 0)

    plsc.subcore_barrier()

    for j in range(NWCH):
        r0 = pl.multiple_of(s * NT + j * WCH, 8)
        o0 = pl.multiple_of(c * NP + r0, 8)
        pltpu.sync_copy(cacc.at[pl.ds(r0, WCH)], z_b)
        pltpu.sync_copy(z_b, cnt_hbm.at[pl.ds(o0, WCH)])


def _p2c(src):
    f = functools.partial(
        pl.kernel,
        out_type=jax.ShapeDtypeStruct((2 * NP, H), jnp.float32),
        mesh=_mesh(),
        scratch_types=[
            pltpu.VMEM((CH,), jnp.int32),
            pltpu.VMEM((CH, H), jnp.float32),
            pltpu.VMEM((WCH, H), jnp.float32),
            pltpu.VMEM_SHARED((NP, H), jnp.float32),
        ],
    )(_p2c_body)
    return f(src)


BE = 2560


def _p3_body(pre, fdp, wfd_t, we2_t, be2, out):
    x = pre[...] + jnp.dot(fdp[...], wfd_t[...],
                           preferred_element_type=jnp.float32)
    e1 = _silu(x)
    y = jnp.dot(e1, we2_t[...], preferred_element_type=jnp.float32) + be2[...]
    out[...] = _silu(y)


def _p3(pre, fdp, wfd_t, we2_t, be2):
    return pl.pallas_call(
        _p3_body,
        grid=(E // BE,),
        in_specs=[
            pl.BlockSpec((BE, H), lambda i: (i, 0)),
            pl.BlockSpec((BE, 8), lambda i: (i, 0)),
            pl.BlockSpec((8, H), lambda i: (0, 0)),
            pl.BlockSpec((H, H), lambda i: (0, 0)),
            pl.BlockSpec((1, H), lambda i: (0, 0)),
        ],
        out_specs=pl.BlockSpec((BE, H), lambda i: (i, 0)),
        out_shape=jax.ShapeDtypeStruct((E, H), jnp.float32),
    )(pre, fdp, wfd_t, we2_t, be2)


def _p4_body(ef_hbm, src_hbm, sum_hbm, idx, buf_e, wout, acc):
    c = lax.axis_index("c")
    s = lax.axis_index("s")
    base = c * (E // 2) + s * ET

    def zrow(r, c2):
        for j in range(H // 16):
            wout[r, pl.ds(j * 16, 16)] = jnp.zeros((16,), jnp.float32)
        return c2

    lax.fori_loop(0, WCH, zrow, 0)

    for j in range(NWCH):
        r0 = pl.multiple_of(s * NT + j * WCH, 8)
        pltpu.sync_copy(wout, acc.at[pl.ds(r0, WCH)])

    plsc.subcore_barrier()

    def chunk(ci, carry):
        off = pl.multiple_of(base + ci * CH, 8)
        pltpu.sync_copy(src_hbm.at[pl.ds(off, CH)], idx)
        pltpu.sync_copy(ef_hbm.at[pl.ds(off, CH)], buf_e)
        pltpu.sync_copy(buf_e, acc.at[idx], add=True)
        return carry

    lax.fori_loop(0, NCH, chunk, 0)

    plsc.subcore_barrier()

    for j in range(NWCH):
        r0 = pl.multiple_of(s * NT + j * WCH, 8)
        o0 = pl.multiple_of(c * NP + r0, 8)
        pltpu.sync_copy(acc.at[pl.ds(r0, WCH)], wout)
        pltpu.sync_copy(wout, sum_hbm.at[pl.ds(o0, WCH)])


def _p4(ef, src):
    f = functools.partial(
        pl.kernel,
        out_type=jax.ShapeDtypeStruct((2 * NP, H), jnp.float32),
        mesh=_mesh(),
        scratch_types=[
            pltpu.VMEM((CH,), jnp.int32),
            pltpu.VMEM((CH, H), jnp.float32),
            pltpu.VMEM((WCH, H), jnp.float32),
            pltpu.VMEM_SHARED((NP, H), jnp.float32),
        ],
    )(_p4_body)
    return f(ef, src)


BN = 2000


def _p5_body(nf, sum0, sum1, cnt0, cnt1, wn1a_t, wn1b_t, bn1, wn2_t, bn2,
             out):
    x = nf[...]
    ssum = sum0[0] + sum1[0]
    cnt = cnt0[0][:, 0:1] + cnt1[0][:, 0:1]
    agg = ssum / jnp.clip(cnt, 1.0, None)
    h = (
        jnp.dot(x, wn1a_t[...], preferred_element_type=jnp.float32)
        + jnp.dot(agg, wn1b_t[...], preferred_element_type=jnp.float32)
        + bn1[...]
    )
    h = _silu(h)
    y = jnp.dot(h, wn2_t[...], preferred_element_type=jnp.float32) + bn2[...]
    out[...] = x + _silu(y)


def _p5(nf, sums, cnts, wn1a_t, wn1b_t, bn1, wn2_t, bn2):
    nb = N // BN
    return pl.pallas_call(
        _p5_body,
        grid=(nb,),
        in_specs=[
            pl.BlockSpec((BN, H), lambda i: (i, 0)),
            pl.BlockSpec((1, BN, H), lambda i: (0, i, 0)),
            pl.BlockSpec((1, BN, H), lambda i: (1, i, 0)),
            pl.BlockSpec((1, BN, H), lambda i: (0, i, 0)),
            pl.BlockSpec((1, BN, H), lambda i: (1, i, 0)),
            pl.BlockSpec((H, H), lambda i: (0, 0)),
            pl.BlockSpec((H, H), lambda i: (0, 0)),
            pl.BlockSpec((1, H), lambda i: (0, 0)),
            pl.BlockSpec((H, H), lambda i: (0, 0)),
            pl.BlockSpec((1, H), lambda i: (0, 0)),
        ],
        out_specs=pl.BlockSpec((BN, H), lambda i: (i, 0)),
        out_shape=jax.ShapeDtypeStruct((N, H), jnp.float32),
    )(nf, sums, sums, cnts, cnts, wn1a_t, wn1b_t, bn1, wn2_t, bn2)


def kernel(node_features, frac_coords, lattices, edge_index, edge2graph,
           frac_diff, We1, be1, We2, be2, Wn1, bn1, Wn2, bn2):
    src = edge_index[0].astype(jnp.int32)
    dst = edge_index[1].astype(jnp.int32)
    g = edge2graph.astype(jnp.int32)

    whi_t = We1[:, :H].T
    whj_t = We1[:, H:2 * H].T
    wlat_t = jnp.pad(We1[:, 2 * H:2 * H + 6].T, ((0, 2), (0, 0)))
    wfd_t = jnp.pad(We1[:, 2 * H + 6:].T, ((0, 5), (0, 0)))
    latp = jnp.pad(lattices, ((0, 0), (0, 2)))
    fdp = jnp.pad(frac_diff, ((0, 0), (0, 5)))

    a, b, c = _p1(node_features, latp, whi_t, whj_t, wlat_t,
                  be1.reshape(1, H))
    pre = _p2(a, b, c, src, dst, g)
    cnts = _p2c(src)
    ef = _p3(pre, fdp, wfd_t, We2.T, be2.reshape(1, H))
    sums = _p4(ef, src)
    sums = sums.reshape(2, NP, H)
    cnts = cnts.reshape(2, NP, H)
    return _p5(node_features, sums, cnts, Wn1[:, :H].T, Wn1[:, H:].T,
               bn1.reshape(1, H), Wn2.T, bn2.reshape(1, H))


# speedup vs baseline: 3.9147x; 1.1133x over previous
import functools

import jax
import jax.numpy as jnp
from jax import lax
from jax.experimental import pallas as pl
from jax.experimental.pallas import tpu as pltpu
from jax.experimental.pallas import tpu_sc as plsc

N = 10000
E = 320000
G = 256
H = 128

NW = 32
ET = E // NW
CH = 80
NCH = ET // CH
NP = 10240
NT = NP // 16
WCH = 128
NWCH = NT // WCH

def _mesh():
    return plsc.VectorSubcoreMesh(core_axis_name="c", subcore_axis_name="s")


def _silu(x):
    return x * jax.nn.sigmoid(x)


def _p1_body(nf, latp, whi_t, whj_t, wlat_t, be1, a_out, b_out, c_out):
    x = nf[...]
    a_out[...] = jnp.dot(x, whi_t[...], preferred_element_type=jnp.float32)
    b_out[...] = jnp.dot(x, whj_t[...], preferred_element_type=jnp.float32)
    c_out[...] = (
        jnp.dot(latp[...], wlat_t[...], preferred_element_type=jnp.float32)
        + be1[...]
    )


def _p1(nf, latp, whi_t, whj_t, wlat_t, be1):
    return pl.pallas_call(
        _p1_body,
        out_shape=(
            jax.ShapeDtypeStruct((N, H), jnp.float32),
            jax.ShapeDtypeStruct((N, H), jnp.float32),
            jax.ShapeDtypeStruct((G, H), jnp.float32),
        ),
    )(nf, latp, whi_t, whj_t, wlat_t, be1)


def _p2_body(a_hbm, b_hbm, c_hbm, src_hbm, dst_hbm, g_hbm, out_hbm,
             isrc, idst, ig, bufs, sems):
    c = lax.axis_index("c")
    s = lax.axis_index("s")
    wid = s * 2 + c
    base = wid * ET

    pltpu.sync_copy(src_hbm.at[pl.ds(pl.multiple_of(base, 8), ET)], isrc)
    pltpu.sync_copy(dst_hbm.at[pl.ds(pl.multiple_of(base, 8), ET)], idst)
    pltpu.sync_copy(g_hbm.at[pl.ds(pl.multiple_of(base, 8), ET)], ig)

    def issue(ci, k):
        lo = pl.multiple_of(ci * CH, 8)
        pltpu.async_copy(a_hbm.at[isrc.at[pl.ds(lo, CH)]], bufs[k][0],
                         sems[k][0])
        pltpu.async_copy(b_hbm.at[idst.at[pl.ds(lo, CH)]], bufs[k][1],
                         sems[k][1])
        pltpu.async_copy(c_hbm.at[ig.at[pl.ds(lo, CH)]], bufs[k][2],
                         sems[k][2])

    def wait(k):
        for j in range(3):
            pltpu.make_async_copy(a_hbm.at[isrc.at[pl.ds(0, CH)]],
                                  bufs[k][j], sems[k][j]).wait()

    def process(ci, k):
        ba, bb, bc = bufs[k]

        def add_row(r, c2):
            for j in range(H // 16):
                sl = pl.ds(j * 16, 16)
                ba[r, sl] = ba[r, sl] + bb[r, sl] + bc[r, sl]
            return c2

        lax.fori_loop(0, CH, add_row, 0)
        off = pl.multiple_of(base + ci * CH, 8)
        pltpu.sync_copy(ba, out_hbm.at[pl.ds(off, CH)])

    issue(0, 0)

    def body2(kk, carry):
        ci = kk * 2
        issue(ci + 1, 1)
        wait(0)
        process(ci, 0)
        issue(ci + 2, 0)
        wait(1)
        process(ci + 1, 1)
        return carry

    lax.fori_loop(0, (NCH - 1) // 2, body2, 0)
    wait(0)
    process(NCH - 1, 0)


def _p2(a, b, c, src, dst, g):
    f = functools.partial(
        pl.kernel,
        out_type=jax.ShapeDtypeStruct((E, H), jnp.float32),
        mesh=_mesh(),
        scratch_types=[
            pltpu.VMEM((ET,), jnp.int32),
            pltpu.VMEM((ET,), jnp.int32),
            pltpu.VMEM((ET,), jnp.int32),
            [[pltpu.VMEM((CH, H), jnp.float32) for _ in range(3)]
             for _ in range(2)],
            [[pltpu.SemaphoreType.DMA for _ in range(3)] for _ in range(2)],
        ],
    )(_p2_body)
    return f(a, b, c, src, dst, g)


def _p2c_body(src3_hbm, cnt_hbm, idx2, ones_b, z_b, cacc):
    c = lax.axis_index("c")
    s = lax.axis_index("s")
    wid = c * 16 + s

    def orow(r, c2):
        for j in range(H // 16):
            ones_b[r, pl.ds(j * 16, 16)] = jnp.ones((16,), jnp.float32)
        return c2

    lax.fori_loop(0, CH, orow, 0)

    def zrow(r, c2):
        for j in range(H // 16):
            z_b[r, pl.ds(j * 16, 16)] = jnp.zeros((16,), jnp.float32)
        return c2

    lax.fori_loop(0, WCH, zrow, 0)

    pltpu.sync_copy(src3_hbm.at[wid], idx2)
    for j in range(NWCH):
        r0 = pl.multiple_of(s * NT + j * WCH, 8)
        pltpu.sync_copy(z_b, cacc.at[pl.ds(r0, WCH)])

    plsc.subcore_barrier()

    def chunk(ci, carry):
        pltpu.sync_copy(ones_b, cacc.at[idx2.at[ci]], add=True)
        return carry

    lax.fori_loop(0, NCH, chunk, 0)

    plsc.subcore_barrier()

    for j in range(NWCH):
        r0 = pl.multiple_of(s * NT + j * WCH, 8)
        o0 = pl.multiple_of(c * NP + r0, 8)
        pltpu.sync_copy(cacc.at[pl.ds(r0, WCH)], z_b)
        pltpu.sync_copy(z_b, cnt_hbm.at[pl.ds(o0, WCH)])


def _p2c(src3):
    f = functools.partial(
        pl.kernel,
        out_type=jax.ShapeDtypeStruct((2 * NP, H), jnp.float32),
        mesh=_mesh(),
        scratch_types=[
            pltpu.VMEM((NCH, CH), jnp.int32),
            pltpu.VMEM((CH, H), jnp.float32),
            pltpu.VMEM((WCH, H), jnp.float32),
            pltpu.VMEM_SHARED((NP, H), jnp.float32),
        ],
    )(_p2c_body)
    return f(src3)


BE = 2560


def _p3_body(pre, fdp, wfd_t, we2_t, be2, out):
    x = pre[...] + jnp.dot(fdp[...], wfd_t[...],
                           preferred_element_type=jnp.float32)
    e1 = _silu(x)
    y = jnp.dot(e1, we2_t[...], preferred_element_type=jnp.float32) + be2[...]
    out[...] = _silu(y)


def _p3(pre, fdp, wfd_t, we2_t, be2):
    return pl.pallas_call(
        _p3_body,
        grid=(E // BE,),
        in_specs=[
            pl.BlockSpec((BE, H), lambda i: (i, 0)),
            pl.BlockSpec((BE, 8), lambda i: (i, 0)),
            pl.BlockSpec((8, H), lambda i: (0, 0)),
            pl.BlockSpec((H, H), lambda i: (0, 0)),
            pl.BlockSpec((1, H), lambda i: (0, 0)),
        ],
        out_specs=pl.BlockSpec((BE, H), lambda i: (i, 0)),
        out_shape=jax.ShapeDtypeStruct((E, H), jnp.float32),
    )(pre, fdp, wfd_t, we2_t, be2)


def _p4_body(ef_hbm, src3_hbm, sum_hbm, idx2, buf0, buf1, acc, se0, se1):
    c = lax.axis_index("c")
    s = lax.axis_index("s")
    wid = c * 16 + s
    base = wid * ET
    bufs = (buf0, buf1)
    sems = (se0, se1)

    pltpu.sync_copy(src3_hbm.at[wid], idx2)

    def zrow(r, c2):
        for j in range(H // 16):
            buf0[r, pl.ds(j * 16, 16)] = jnp.zeros((16,), jnp.float32)
        return c2

    lax.fori_loop(0, CH, zrow, 0)

    for j in range(NT // CH):
        r0 = pl.multiple_of(s * NT + j * CH, 8)
        pltpu.sync_copy(buf0, acc.at[pl.ds(r0, CH)])

    plsc.subcore_barrier()

    def issue(ci, k):
        off = pl.multiple_of(base + ci * CH, 8)
        pltpu.async_copy(ef_hbm.at[pl.ds(off, CH)], bufs[k], sems[k])

    def wait(k):
        pltpu.make_async_copy(ef_hbm.at[pl.ds(0, CH)], bufs[k],
                              sems[k]).wait()

    def scat(ci, k):
        pltpu.sync_copy(bufs[k], acc.at[idx2.at[ci]], add=True)

    issue(0, 0)

    def body2(kk, carry):
        ci = kk * 2
        issue(ci + 1, 1)
        wait(0)
        scat(ci, 0)
        issue(ci + 2, 0)
        wait(1)
        scat(ci + 1, 1)
        return carry

    lax.fori_loop(0, (NCH - 1) // 2, body2, 0)
    wait(0)
    scat(NCH - 1, 0)

    plsc.subcore_barrier()

    for j in range(NT // CH):
        r0 = pl.multiple_of(s * NT + j * CH, 8)
        o0 = pl.multiple_of(c * NP + r0, 8)
        pltpu.sync_copy(acc.at[pl.ds(r0, CH)], buf0)
        pltpu.sync_copy(buf0, sum_hbm.at[pl.ds(o0, CH)])


def _p4(ef, src3):
    f = functools.partial(
        pl.kernel,
        out_type=jax.ShapeDtypeStruct((2 * NP, H), jnp.float32),
        mesh=_mesh(),
        scratch_types=[
            pltpu.VMEM((NCH, CH), jnp.int32),
            pltpu.VMEM((CH, H), jnp.float32),
            pltpu.VMEM((CH, H), jnp.float32),
            pltpu.VMEM_SHARED((NP, H), jnp.float32),
            pltpu.SemaphoreType.DMA,
            pltpu.SemaphoreType.DMA,
        ],
    )(_p4_body)
    return f(ef, src3)


BN = 2000


def _p5_body(nf, sum0, sum1, cnt0, cnt1, wn1a_t, wn1b_t, bn1, wn2_t, bn2,
             out):
    x = nf[...]
    ssum = sum0[0] + sum1[0]
    cnt = cnt0[0][:, 0:1] + cnt1[0][:, 0:1]
    agg = ssum / jnp.clip(cnt, 1.0, None)
    h = (
        jnp.dot(x, wn1a_t[...], preferred_element_type=jnp.float32)
        + jnp.dot(agg, wn1b_t[...], preferred_element_type=jnp.float32)
        + bn1[...]
    )
    h = _silu(h)
    y = jnp.dot(h, wn2_t[...], preferred_element_type=jnp.float32) + bn2[...]
    out[...] = x + _silu(y)


def _p5(nf, sums, cnts, wn1a_t, wn1b_t, bn1, wn2_t, bn2):
    nb = N // BN
    return pl.pallas_call(
        _p5_body,
        grid=(nb,),
        in_specs=[
            pl.BlockSpec((BN, H), lambda i: (i, 0)),
            pl.BlockSpec((1, BN, H), lambda i: (0, i, 0)),
            pl.BlockSpec((1, BN, H), lambda i: (1, i, 0)),
            pl.BlockSpec((1, BN, H), lambda i: (0, i, 0)),
            pl.BlockSpec((1, BN, H), lambda i: (1, i, 0)),
            pl.BlockSpec((H, H), lambda i: (0, 0)),
            pl.BlockSpec((H, H), lambda i: (0, 0)),
            pl.BlockSpec((1, H), lambda i: (0, 0)),
            pl.BlockSpec((H, H), lambda i: (0, 0)),
            pl.BlockSpec((1, H), lambda i: (0, 0)),
        ],
        out_specs=pl.BlockSpec((BN, H), lambda i: (i, 0)),
        out_shape=jax.ShapeDtypeStruct((N, H), jnp.float32),
    )(nf, sums, sums, cnts, cnts, wn1a_t, wn1b_t, bn1, wn2_t, bn2)


def kernel(node_features, frac_coords, lattices, edge_index, edge2graph,
           frac_diff, We1, be1, We2, be2, Wn1, bn1, Wn2, bn2):
    src = edge_index[0].astype(jnp.int32)
    dst = edge_index[1].astype(jnp.int32)
    g = edge2graph.astype(jnp.int32)

    whi_t = We1[:, :H].T
    whj_t = We1[:, H:2 * H].T
    wlat_t = jnp.pad(We1[:, 2 * H:2 * H + 6].T, ((0, 2), (0, 0)))
    wfd_t = jnp.pad(We1[:, 2 * H + 6:].T, ((0, 5), (0, 0)))
    latp = jnp.pad(lattices, ((0, 0), (0, 2)))
    fdp = jnp.pad(frac_diff, ((0, 0), (0, 5)))

    a, b, c = _p1(node_features, latp, whi_t, whj_t, wlat_t,
                  be1.reshape(1, H))
    src3 = src.reshape(NW, NCH, CH)
    pre = _p2(a, b, c, src, dst, g)
    cnts = _p2c(src3)
    ef = _p3(pre, fdp, wfd_t, We2.T, be2.reshape(1, H))
    sums = _p4(ef, src3)
    sums = sums.reshape(2, NP, H)
    cnts = cnts.reshape(2, NP, H)
    return _p5(node_features, sums, cnts, Wn1[:, :H].T, Wn1[:, H:].T,
               bn1.reshape(1, H), Wn2.T, bn2.reshape(1, H))


# speedup vs baseline: 4.0929x; 1.0455x over previous
import functools

import jax
import jax.numpy as jnp
from jax import lax
from jax.experimental import pallas as pl
from jax.experimental.pallas import tpu as pltpu
from jax.experimental.pallas import tpu_sc as plsc

N = 10000
E = 320000
G = 256
H = 128

NW = 32
ET = E // NW
CH = 80
NCH = ET // CH
NP = 10240
NT = NP // 16
WCH = 128
NWCH = NT // WCH

def _mesh():
    return plsc.VectorSubcoreMesh(core_axis_name="c", subcore_axis_name="s")


def _silu(x):
    return x * jax.nn.sigmoid(x)


def _p1_body(nf, latp, whi_t, whj_t, wlat_t, be1, a_out, b_out, c_out):
    x = nf[...]
    a_out[...] = jnp.dot(x, whi_t[...], preferred_element_type=jnp.float32)
    b_out[...] = jnp.dot(x, whj_t[...], preferred_element_type=jnp.float32)
    c_out[...] = (
        jnp.dot(latp[...], wlat_t[...], preferred_element_type=jnp.float32)
        + be1[...]
    )


def _p1(nf, latp, whi_t, whj_t, wlat_t, be1):
    return pl.pallas_call(
        _p1_body,
        out_shape=(
            jax.ShapeDtypeStruct((N, H), jnp.float32),
            jax.ShapeDtypeStruct((N, H), jnp.float32),
            jax.ShapeDtypeStruct((G, H), jnp.float32),
        ),
    )(nf, latp, whi_t, whj_t, wlat_t, be1)


def _p2_body(a_hbm, b_hbm, src_hbm, dst_hbm, out_hbm,
             isrc, idst, bufs, outs, gsems, wsems):
    c = lax.axis_index("c")
    s = lax.axis_index("s")
    wid = s * 2 + c
    base = wid * ET

    pltpu.sync_copy(src_hbm.at[pl.ds(pl.multiple_of(base, 8), ET)], isrc)
    pltpu.sync_copy(dst_hbm.at[pl.ds(pl.multiple_of(base, 8), ET)], idst)

    def issue(ci, k):
        lo = pl.multiple_of(ci * CH, 8)
        pltpu.async_copy(a_hbm.at[isrc.at[pl.ds(lo, CH)]], bufs[k][0],
                         gsems[k][0])
        pltpu.async_copy(b_hbm.at[idst.at[pl.ds(lo, CH)]], bufs[k][1],
                         gsems[k][1])

    def wait_gather(k):
        for j in range(2):
            pltpu.make_async_copy(a_hbm.at[isrc.at[pl.ds(0, CH)]],
                                  bufs[k][j], gsems[k][j]).wait()

    def wait_write(k):
        pltpu.make_async_copy(outs[k], out_hbm.at[pl.ds(0, CH)],
                              wsems[k]).wait()

    def process(ci, k):
        ba, bb = bufs[k]
        bo = outs[k]

        def add_row(r, c2):
            for j in range(H // 16):
                sl = pl.ds(j * 16, 16)
                bo[r, sl] = ba[r, sl] + bb[r, sl]
            return c2

        lax.fori_loop(0, CH, add_row, 0)
        off = pl.multiple_of(base + ci * CH, 8)
        pltpu.async_copy(bo, out_hbm.at[pl.ds(off, CH)], wsems[k])

    issue(0, 0)

    def body2(kk, carry):
        ci = kk * 2
        issue(ci + 1, 1)
        wait_gather(0)

        @pl.when(kk > 0)
        def _():
            wait_write(0)

        process(ci, 0)
        issue(ci + 2, 0)
        wait_gather(1)

        @pl.when(kk > 0)
        def _():
            wait_write(1)

        process(ci + 1, 1)
        return carry

    lax.fori_loop(0, (NCH - 1) // 2, body2, 0)
    wait_gather(0)
    wait_write(0)
    process(NCH - 1, 0)
    wait_write(0)
    wait_write(1)


def _p2(a, b, src, dst):
    f = functools.partial(
        pl.kernel,
        out_type=jax.ShapeDtypeStruct((E, H), jnp.float32),
        mesh=_mesh(),
        scratch_types=[
            pltpu.VMEM((ET,), jnp.int32),
            pltpu.VMEM((ET,), jnp.int32),
            [[pltpu.VMEM((CH, H), jnp.float32) for _ in range(2)]
             for _ in range(2)],
            [pltpu.VMEM((CH, H), jnp.float32) for _ in range(2)],
            [[pltpu.SemaphoreType.DMA for _ in range(2)] for _ in range(2)],
            [pltpu.SemaphoreType.DMA for _ in range(2)],
        ],
    )(_p2_body)
    return f(a, b, src, dst)


def _p2c_body(src3_hbm, cnt_hbm, idx2, ones_b, z_b, cacc):
    c = lax.axis_index("c")
    s = lax.axis_index("s")
    wid = c * 16 + s

    def orow(r, c2):
        for j in range(H // 16):
            ones_b[r, pl.ds(j * 16, 16)] = jnp.ones((16,), jnp.float32)
        return c2

    lax.fori_loop(0, CH, orow, 0)

    def zrow(r, c2):
        for j in range(H // 16):
            z_b[r, pl.ds(j * 16, 16)] = jnp.zeros((16,), jnp.float32)
        return c2

    lax.fori_loop(0, WCH, zrow, 0)

    pltpu.sync_copy(src3_hbm.at[wid], idx2)
    for j in range(NWCH):
        r0 = pl.multiple_of(s * NT + j * WCH, 8)
        pltpu.sync_copy(z_b, cacc.at[pl.ds(r0, WCH)])

    plsc.subcore_barrier()

    def chunk(ci, carry):
        pltpu.sync_copy(ones_b, cacc.at[idx2.at[ci]], add=True)
        return carry

    lax.fori_loop(0, NCH, chunk, 0)

    plsc.subcore_barrier()

    for j in range(NWCH):
        r0 = pl.multiple_of(s * NT + j * WCH, 8)
        o0 = pl.multiple_of(c * NP + r0, 8)
        pltpu.sync_copy(cacc.at[pl.ds(r0, WCH)], z_b)
        pltpu.sync_copy(z_b, cnt_hbm.at[pl.ds(o0, WCH)])


def _p2c(src3):
    f = functools.partial(
        pl.kernel,
        out_type=jax.ShapeDtypeStruct((2 * NP, H), jnp.float32),
        mesh=_mesh(),
        scratch_types=[
            pltpu.VMEM((NCH, CH), jnp.int32),
            pltpu.VMEM((CH, H), jnp.float32),
            pltpu.VMEM((WCH, H), jnp.float32),
            pltpu.VMEM_SHARED((NP, H), jnp.float32),
        ],
    )(_p2c_body)
    return f(src3)


BE = 2560


def _p3_body(pre, g2, cmat, fdp, wfd_t, we2_t, be2, out):
    oh = (jax.lax.broadcasted_iota(jnp.int32, (BE, G), 1)
          == g2[...]).astype(jnp.float32)
    x = (pre[...]
         + jnp.dot(oh, cmat[...], preferred_element_type=jnp.float32)
         + jnp.dot(fdp[...], wfd_t[...], preferred_element_type=jnp.float32))
    e1 = _silu(x)
    y = jnp.dot(e1, we2_t[...], preferred_element_type=jnp.float32) + be2[...]
    out[...] = _silu(y)


def _p3(pre, g2, cmat, fdp, wfd_t, we2_t, be2):
    return pl.pallas_call(
        _p3_body,
        grid=(E // BE,),
        in_specs=[
            pl.BlockSpec((BE, H), lambda i: (i, 0)),
            pl.BlockSpec((BE, 1), lambda i: (i, 0)),
            pl.BlockSpec((G, H), lambda i: (0, 0)),
            pl.BlockSpec((BE, 8), lambda i: (i, 0)),
            pl.BlockSpec((8, H), lambda i: (0, 0)),
            pl.BlockSpec((H, H), lambda i: (0, 0)),
            pl.BlockSpec((1, H), lambda i: (0, 0)),
        ],
        out_specs=pl.BlockSpec((BE, H), lambda i: (i, 0)),
        out_shape=jax.ShapeDtypeStruct((E, H), jnp.float32),
    )(pre, g2, cmat, fdp, wfd_t, we2_t, be2)


def _p4_body(ef_hbm, src3_hbm, sum_hbm, idx2, buf0, buf1, acc, se0, se1):
    c = lax.axis_index("c")
    s = lax.axis_index("s")
    wid = c * 16 + s
    base = wid * ET
    bufs = (buf0, buf1)
    sems = (se0, se1)

    pltpu.sync_copy(src3_hbm.at[wid], idx2)

    def zrow(r, c2):
        for j in range(H // 16):
            buf0[r, pl.ds(j * 16, 16)] = jnp.zeros((16,), jnp.float32)
        return c2

    lax.fori_loop(0, CH, zrow, 0)

    for j in range(NT // CH):
        r0 = pl.multiple_of(s * NT + j * CH, 8)
        pltpu.sync_copy(buf0, acc.at[pl.ds(r0, CH)])

    plsc.subcore_barrier()

    def issue(ci, k):
        off = pl.multiple_of(base + ci * CH, 8)
        pltpu.async_copy(ef_hbm.at[pl.ds(off, CH)], bufs[k], sems[k])

    def wait(k):
        pltpu.make_async_copy(ef_hbm.at[pl.ds(0, CH)], bufs[k],
                              sems[k]).wait()

    def scat(ci, k):
        pltpu.sync_copy(bufs[k], acc.at[idx2.at[ci]], add=True)

    issue(0, 0)

    def body2(kk, carry):
        ci = kk * 2
        issue(ci + 1, 1)
        wait(0)
        scat(ci, 0)
        issue(ci + 2, 0)
        wait(1)
        scat(ci + 1, 1)
        return carry

    lax.fori_loop(0, (NCH - 1) // 2, body2, 0)
    wait(0)
    scat(NCH - 1, 0)

    plsc.subcore_barrier()

    for j in range(NT // CH):
        r0 = pl.multiple_of(s * NT + j * CH, 8)
        o0 = pl.multiple_of(c * NP + r0, 8)
        pltpu.sync_copy(acc.at[pl.ds(r0, CH)], buf0)
        pltpu.sync_copy(buf0, sum_hbm.at[pl.ds(o0, CH)])


def _p4(ef, src3):
    f = functools.partial(
        pl.kernel,
        out_type=jax.ShapeDtypeStruct((2 * NP, H), jnp.float32),
        mesh=_mesh(),
        scratch_types=[
            pltpu.VMEM((NCH, CH), jnp.int32),
            pltpu.VMEM((CH, H), jnp.float32),
            pltpu.VMEM((CH, H), jnp.float32),
            pltpu.VMEM_SHARED((NP, H), jnp.float32),
            pltpu.SemaphoreType.DMA,
            pltpu.SemaphoreType.DMA,
        ],
    )(_p4_body)
    return f(ef, src3)


BN = 2000


def _p5_body(nf, sum0, sum1, cnt0, cnt1, wn1a_t, wn1b_t, bn1, wn2_t, bn2,
             out):
    x = nf[...]
    ssum = sum0[0] + sum1[0]
    cnt = cnt0[0][:, 0:1] + cnt1[0][:, 0:1]
    agg = ssum / jnp.clip(cnt, 1.0, None)
    h = (
        jnp.dot(x, wn1a_t[...], preferred_element_type=jnp.float32)
        + jnp.dot(agg, wn1b_t[...], preferred_element_type=jnp.float32)
        + bn1[...]
    )
    h = _silu(h)
    y = jnp.dot(h, wn2_t[...], preferred_element_type=jnp.float32) + bn2[...]
    out[...] = x + _silu(y)


def _p5(nf, sums, cnts, wn1a_t, wn1b_t, bn1, wn2_t, bn2):
    nb = N // BN
    return pl.pallas_call(
        _p5_body,
        grid=(nb,),
        in_specs=[
            pl.BlockSpec((BN, H), lambda i: (i, 0)),
            pl.BlockSpec((1, BN, H), lambda i: (0, i, 0)),
            pl.BlockSpec((1, BN, H), lambda i: (1, i, 0)),
            pl.BlockSpec((1, BN, H), lambda i: (0, i, 0)),
            pl.BlockSpec((1, BN, H), lambda i: (1, i, 0)),
            pl.BlockSpec((H, H), lambda i: (0, 0)),
            pl.BlockSpec((H, H), lambda i: (0, 0)),
            pl.BlockSpec((1, H), lambda i: (0, 0)),
            pl.BlockSpec((H, H), lambda i: (0, 0)),
            pl.BlockSpec((1, H), lambda i: (0, 0)),
        ],
        out_specs=pl.BlockSpec((BN, H), lambda i: (i, 0)),
        out_shape=jax.ShapeDtypeStruct((N, H), jnp.float32),
    )(nf, sums, sums, cnts, cnts, wn1a_t, wn1b_t, bn1, wn2_t, bn2)


def kernel(node_features, frac_coords, lattices, edge_index, edge2graph,
           frac_diff, We1, be1, We2, be2, Wn1, bn1, Wn2, bn2):
    src = edge_index[0].astype(jnp.int32)
    dst = edge_index[1].astype(jnp.int32)
    g = edge2graph.astype(jnp.int32)

    whi_t = We1[:, :H].T
    whj_t = We1[:, H:2 * H].T
    wlat_t = jnp.pad(We1[:, 2 * H:2 * H + 6].T, ((0, 2), (0, 0)))
    wfd_t = jnp.pad(We1[:, 2 * H + 6:].T, ((0, 5), (0, 0)))
    latp = jnp.pad(lattices, ((0, 0), (0, 2)))
    fdp = jnp.pad(frac_diff, ((0, 0), (0, 5)))

    a, b, c = _p1(node_features, latp, whi_t, whj_t, wlat_t,
                  be1.reshape(1, H))
    src3 = src.reshape(NW, NCH, CH)
    pre = _p2(a, b, src, dst)
    cnts = _p2c(src3)
    ef = _p3(pre, g.reshape(E, 1), c, fdp, wfd_t, We2.T, be2.reshape(1, H))
    sums = _p4(ef, src3)
    sums = sums.reshape(2, NP, H)
    cnts = cnts.reshape(2, NP, H)
    return _p5(node_features, sums, cnts, Wn1[:, :H].T, Wn1[:, H:].T,
               bn1.reshape(1, H), Wn2.T, bn2.reshape(1, H))
